# baseline (speedup 1.0000x reference)
import jax
import jax.numpy as jnp
from jax import lax
from jax.experimental import pallas as pl
from jax.experimental.pallas import tpu as pltpu
from jax.experimental.pallas import tpu_sc as plsc

_D = 1024
_CTX = 2048
_NC, _NS, _L = 2, 16, 16
_NW = _NC * _NS
_VPR = _D // _L
_NBUF = 3


def _make_emb_kernel(batch: int):
    tpw = _CTX // _NW
    chunk = 16
    nchunk = batch * tpw // chunk
    hpb = tpw // chunk

    mesh = plsc.VectorSubcoreMesh(core_axis_name="c", subcore_axis_name="s")

    @pl.kernel(
        out_type=jax.ShapeDtypeStruct((batch, _CTX, _D), jnp.float32),
        mesh=mesh,
        scratch_types=[
            pltpu.VMEM((batch * tpw,), jnp.int32),
            pltpu.VMEM((_NBUF, chunk, _D), jnp.float32),
            pltpu.VMEM((tpw, _D), jnp.float32),
            [pltpu.SemaphoreType.DMA] * _NBUF,
            [pltpu.SemaphoreType.DMA] * _NBUF,
            pltpu.SemaphoreType.DMA,
            pltpu.SemaphoreType.DMA,
        ],
    )
    def emb(x_hbm, w_hbm, pos_hbm, out_hbm,
            idx_v, rows_v, pos_v, gsems, ssems, psem, isem):
        wid = lax.axis_index("s") * _NC + lax.axis_index("c")
        tbase = wid * tpw

        pos_cp = pltpu.async_copy(pos_hbm.at[pl.ds(tbase, tpw)], pos_v, psem)
        idx_cps = [
            pltpu.async_copy(x_hbm.at[b, pl.ds(tbase, tpw)],
                             idx_v.at[pl.ds(b * tpw, tpw)], isem)
            for b in range(batch)
        ]
        for cp in idx_cps:
            cp.wait()

        def gstart(c):
            b, h = c // hpb, c % hpb
            return pltpu.async_copy(
                w_hbm.at[idx_v.at[pl.ds(b * tpw + h * chunk, chunk)]],
                rows_v.at[c % _NBUF], gsems[c % _NBUF])

        def sstart(c):
            b, h = c // hpb, c % hpb
            return pltpu.async_copy(
                rows_v.at[c % _NBUF],
                out_hbm.at[b, pl.ds(tbase + h * chunk, chunk)],
                ssems[c % _NBUF])

        gcp = {c: gstart(c) for c in range(_NBUF - 1)}
        scp = {}
        pos_cp.wait()
        for c in range(nchunk):
            n = c + _NBUF - 1
            if n < nchunk:
                if n - _NBUF >= 0:
                    scp[n - _NBUF].wait()
                gcp[n] = gstart(n)
            gcp[c].wait()
            buf = rows_v.at[c % _NBUF]
            poff = (c % hpb) * chunk

            @plsc.parallel_loop(0, chunk * 2)
            def _add(r2):
                r = r2 >> 1
                k0 = pl.multiple_of((r2 & 1) << 9, _L)
                for k in range(_VPR // 2):
                    plsc.addupdate(buf.at[r, pl.ds(k0 + k * _L, _L)],
                                   pos_v[poff + r, pl.ds(k0 + k * _L, _L)])

            scp[c] = sstart(c)
        for c in range(nchunk - _NBUF, nchunk):
            if c >= 0 and c in scp:
                scp[c].wait()

    return emb


def kernel(x, W, pos):
    x = x.reshape(-1, _CTX)
    return _make_emb_kernel(x.shape[0])(x, W, pos)


# speedup vs baseline: 1.1842x; 1.1842x over previous
import jax
import jax.numpy as jnp
from jax import lax
from jax.experimental import pallas as pl
from jax.experimental.pallas import tpu as pltpu
from jax.experimental.pallas import tpu_sc as plsc

_D = 1024
_CTX = 2048
_NC, _NS, _L = 2, 16, 16
_NW = _NC * _NS
_VPR = _D // _L
_NBUF = 3


def _make_emb_kernel(batch: int):
    tpw = _CTX // _NW
    chunk = 16
    nchunk = batch * tpw // chunk
    hpb = tpw // chunk

    mesh = plsc.VectorSubcoreMesh(core_axis_name="c", subcore_axis_name="s")

    @pl.kernel(
        out_type=jax.ShapeDtypeStruct((batch, _CTX, _D), jnp.float32),
        mesh=mesh,
        scratch_types=[
            pltpu.VMEM((batch * tpw,), jnp.int32),
            pltpu.VMEM((_NBUF, chunk, _D), jnp.float32),
            pltpu.VMEM((tpw, _D), jnp.float32),
            [pltpu.SemaphoreType.DMA] * _NBUF,
            [pltpu.SemaphoreType.DMA] * _NBUF,
            pltpu.SemaphoreType.DMA,
            pltpu.SemaphoreType.DMA,
        ],
    )
    def emb(x_hbm, w_hbm, pos_hbm, out_hbm,
            idx_v, rows_v, pos_v, gsems, ssems, psem, isem):
        wid = lax.axis_index("s") * _NC + lax.axis_index("c")
        tbase = wid * tpw

        pos_cp = pltpu.async_copy(pos_hbm.at[pl.ds(tbase, tpw)], pos_v, psem)
        idx_cps = [
            pltpu.async_copy(x_hbm.at[b, pl.ds(tbase, tpw)],
                             idx_v.at[pl.ds(b * tpw, tpw)], isem)
            for b in range(batch)
        ]
        for cp in idx_cps:
            cp.wait()

        def gstart(c):
            b, h = c // hpb, c % hpb
            return pltpu.async_copy(
                w_hbm.at[idx_v.at[pl.ds(b * tpw + h * chunk, chunk)]],
                rows_v.at[c % _NBUF], gsems[c % _NBUF])

        def sstart(c):
            b, h = c // hpb, c % hpb
            return pltpu.async_copy(
                rows_v.at[c % _NBUF],
                out_hbm.at[b, pl.ds(tbase + h * chunk, chunk)],
                ssems[c % _NBUF])

        gcp = {c: gstart(c) for c in range(_NBUF - 1)}
        scp = {}
        pos_cp.wait()
        for c in range(nchunk):
            n = c + _NBUF - 1
            if n < nchunk:
                if n - _NBUF >= 0:
                    scp[n - _NBUF].wait()
                gcp[n] = gstart(n)
            gcp[c].wait()
            buf = rows_v.at[c % _NBUF]
            poff = (c % hpb) * chunk

            @plsc.parallel_loop(0, chunk * _VPR, unroll=8)
            def _add(i):
                r = i >> 6
                k = pl.multiple_of((i & (_VPR - 1)) << 4, _L)
                plsc.addupdate(buf.at[r, pl.ds(k, _L)],
                               pos_v[poff + r, pl.ds(k, _L)])

            scp[c] = sstart(c)
        for c in range(nchunk - _NBUF, nchunk):
            if c >= 0 and c in scp:
                scp[c].wait()

    return emb


def kernel(x, W, pos):
    x = x.reshape(-1, _CTX)
    return _make_emb_kernel(x.shape[0])(x, W, pos)
